# baseline (speedup 1.0000x reference)
import jax
import jax.numpy as jnp
from jax.experimental import pallas as pl
from jax.experimental.pallas import tpu as pltpu

N = 100000
D_IN = 128
D_OUT = 20
EPS = 1e-5
BLK = 5000
NB = N // BLK


def _body(x_ref, w_ref, b_ref, g_ref, be_ref, eye_ref, o_ref,
          h_ref, s1, s2, sc, sh):
    i = pl.program_id(0)

    @pl.when(i == 0)
    def _init():
        s1[...] = jnp.zeros_like(s1)
        s2[...] = jnp.zeros_like(s2)

    @pl.when(i < NB)
    def _sweep1():
        z = jax.lax.dot_general(
            w_ref[...], x_ref[...],
            (((1,), (1,)), ((), ())),
            preferred_element_type=jnp.float32,
            precision=jax.lax.Precision.HIGHEST,
        )
        h = jnp.tanh(z + b_ref[...])
        h_ref[i] = h
        s1[...] += jnp.sum(h, axis=1, keepdims=True)
        s2[...] += jnp.sum(h * h, axis=1, keepdims=True)

    @pl.when(i == NB - 1)
    def _stats():
        mean = s1[...] * (1.0 / N)
        var = s2[...] * (1.0 / N) - mean * mean
        inv = jax.lax.rsqrt(var + EPS) * g_ref[...]
        sc[...] = inv
        sh[...] = be_ref[...] - mean * inv

    @pl.when(i >= NB)
    def _sweep2():
        j = i - NB
        y = h_ref[j] * sc[...] + sh[...]
        o_ref[...] = jax.lax.dot_general(
            y, eye_ref[...],
            (((0,), (0,)), ((), ())),
            preferred_element_type=jnp.float32,
            precision=jax.lax.Precision.HIGHEST,
        )


def kernel(x, W, b, gamma, beta):
    b2 = b.reshape(D_OUT, 1)
    g2 = gamma.reshape(D_OUT, 1)
    be2 = beta.reshape(D_OUT, 1)
    eye = jnp.eye(D_OUT, dtype=jnp.float32)
    return pl.pallas_call(
        _body,
        grid=(2 * NB,),
        in_specs=[
            pl.BlockSpec((BLK, D_IN), lambda i: (jnp.minimum(i, NB - 1), 0)),
            pl.BlockSpec((D_OUT, D_IN), lambda i: (0, 0)),
            pl.BlockSpec((D_OUT, 1), lambda i: (0, 0)),
            pl.BlockSpec((D_OUT, 1), lambda i: (0, 0)),
            pl.BlockSpec((D_OUT, 1), lambda i: (0, 0)),
            pl.BlockSpec((D_OUT, D_OUT), lambda i: (0, 0)),
        ],
        out_specs=pl.BlockSpec(
            (BLK, D_OUT), lambda i: (jnp.where(i < NB, 0, i - NB), 0)
        ),
        out_shape=jax.ShapeDtypeStruct((N, D_OUT), jnp.float32),
        scratch_shapes=[
            pltpu.VMEM((NB, D_OUT, BLK), jnp.float32),
            pltpu.VMEM((D_OUT, 1), jnp.float32),
            pltpu.VMEM((D_OUT, 1), jnp.float32),
            pltpu.VMEM((D_OUT, 1), jnp.float32),
            pltpu.VMEM((D_OUT, 1), jnp.float32),
        ],
    )(x, W, b2, g2, be2, eye)


# speedup vs baseline: 2.1555x; 2.1555x over previous
import jax
import jax.numpy as jnp
from jax.experimental import pallas as pl
from jax.experimental.pallas import tpu as pltpu

N = 100000
D_IN = 128
D_OUT = 20
EPS = 1e-5
BLK = 5000
NB = N // BLK


def _body(x_ref, w_ref, b_ref, g_ref, be_ref, o_ref, h_ref, s1, s2, sc, sh):
    i = pl.program_id(0)

    @pl.when(i == 0)
    def _init():
        s1[...] = jnp.zeros_like(s1)
        s2[...] = jnp.zeros_like(s2)

    @pl.when(i < NB)
    def _sweep1():
        z = jax.lax.dot_general(
            x_ref[...], w_ref[...],
            (((1,), (1,)), ((), ())),
            preferred_element_type=jnp.float32,
        )
        h = jnp.tanh(z + b_ref[...])
        s1[...] += jnp.sum(h, axis=0, keepdims=True)
        s2[...] += jnp.sum(h * h, axis=0, keepdims=True)
        h_ref[i] = h.astype(jnp.bfloat16)

    @pl.when(i == NB - 1)
    def _stats():
        mean = s1[...] * (1.0 / N)
        var = s2[...] * (1.0 / N) - mean * mean
        inv = jax.lax.rsqrt(var + EPS) * g_ref[...]
        sc[...] = inv
        sh[...] = be_ref[...] - mean * inv

    @pl.when(i >= NB)
    def _sweep2():
        j = i - NB
        o_ref[...] = h_ref[j].astype(jnp.float32) * sc[...] + sh[...]


def kernel(x, W, b, gamma, beta):
    b2 = b.reshape(1, D_OUT)
    g2 = gamma.reshape(1, D_OUT)
    be2 = beta.reshape(1, D_OUT)
    return pl.pallas_call(
        _body,
        grid=(2 * NB,),
        in_specs=[
            pl.BlockSpec((BLK, D_IN), lambda i: (jnp.minimum(i, NB - 1), 0)),
            pl.BlockSpec((D_OUT, D_IN), lambda i: (0, 0)),
            pl.BlockSpec((1, D_OUT), lambda i: (0, 0)),
            pl.BlockSpec((1, D_OUT), lambda i: (0, 0)),
            pl.BlockSpec((1, D_OUT), lambda i: (0, 0)),
        ],
        out_specs=pl.BlockSpec(
            (BLK, D_OUT), lambda i: (jnp.where(i < NB, 0, i - NB), 0)
        ),
        out_shape=jax.ShapeDtypeStruct((N, D_OUT), jnp.float32),
        scratch_shapes=[
            pltpu.VMEM((NB, BLK, D_OUT), jnp.bfloat16),
            pltpu.VMEM((1, D_OUT), jnp.float32),
            pltpu.VMEM((1, D_OUT), jnp.float32),
            pltpu.VMEM((1, D_OUT), jnp.float32),
            pltpu.VMEM((1, D_OUT), jnp.float32),
        ],
    )(x, W, b2, g2, be2)
